# Initial kernel scaffold; baseline (speedup 1.0000x reference)
import jax
import jax.numpy as jnp
from jax.experimental import pallas as pl


def kernel(x, pos, edge_index, batch, W_init, W_msg_tp_0, W_msg_lin_0, W_upd_tp_0, W_upd_lin_0, W_msg_tp_1, W_msg_lin_1, W_upd_tp_1, W_upd_lin_1, W_final):
    raise NotImplementedError("write your pallas kernel here")


# speedup vs baseline: 1.8606x; 1.8606x over previous
import functools
import jax
import jax.numpy as jnp
from jax import lax
from jax.experimental import pallas as pl
from jax.experimental.pallas import tpu as pltpu
from jax.experimental.pallas import tpu_sc as plsc

N = 10000
E = 160000
DIN = 128
H = 64
G = 16
LATENT = 128

NC = 2
NS = 16
EPC = E // NC
EPT = EPC // NS
B = 40
NB = EPT // B


def _init_body(x_ref, w_ref, o_ref):
    o_ref[...] = jnp.dot(x_ref[...], w_ref[...],
                         preferred_element_type=jnp.float32) / jnp.sqrt(128.0)


def _pq_body(h_ref, wtp_ref, wlin_ref, o_ref):
    h = h_ref[...]
    p = jnp.dot(h, wtp_ref[...], preferred_element_type=jnp.float32) / 8.0
    q = jnp.dot(h, wlin_ref[...], preferred_element_type=jnp.float32) / 8.0
    o_ref[...] = jnp.concatenate([p, q], axis=1)


def _sqrt_body(s_ref, o_ref):
    o_ref[...] = jnp.sqrt(s_ref[...])


def _upd_body(h_ref, agg2_ref, wt_ref, wlin_ref, o_ref):
    h = h_ref[...]
    agg = agg2_ref[0][:, :H] + agg2_ref[1][:, :H]
    acc = h + jnp.dot(h, wlin_ref[...], preferred_element_type=jnp.float32) / 8.0
    aggs = agg / 64.0
    for j in range(H):
        hw = jnp.dot(h, wt_ref[j], preferred_element_type=jnp.float32)
        acc = acc + aggs[:, j:j + 1] * hw
    o_ref[...] = acc


def _final_body(h_ref, b_ref, wf_ref, o_ref):
    h = h_ref[...]
    b = b_ref[...]
    seg = jax.lax.broadcasted_iota(jnp.int32, (G, N), 0)
    mask = (b == seg).astype(jnp.float32)
    sums = jnp.dot(mask, h, preferred_element_type=jnp.float32)
    counts = jnp.sum(mask, axis=1, keepdims=True)
    mean = sums / jnp.maximum(counts, 1.0)
    o_ref[...] = jnp.dot(mean, wf_ref[...],
                         preferred_element_type=jnp.float32) / 8.0


_init_call = pl.pallas_call(
    _init_body, out_shape=jax.ShapeDtypeStruct((N, H), jnp.float32))

_pq_call = pl.pallas_call(
    _pq_body, out_shape=jax.ShapeDtypeStruct((N, 2 * H), jnp.float32))

_sqrt_call = pl.pallas_call(
    _sqrt_body,
    grid=(20,),
    in_specs=[pl.BlockSpec((E // 20, 16), lambda i: (i, 0))],
    out_specs=pl.BlockSpec((E // 20, 16), lambda i: (i, 0)),
    out_shape=jax.ShapeDtypeStruct((E, 16), jnp.float32))

_upd_call = pl.pallas_call(
    _upd_body,
    grid=(5,),
    in_specs=[
        pl.BlockSpec((N // 5, H), lambda i: (i, 0)),
        pl.BlockSpec((NC, N // 5, 128), lambda i: (0, i, 0)),
        pl.BlockSpec((H, H, H), lambda i: (0, 0, 0)),
        pl.BlockSpec((H, H), lambda i: (0, 0)),
    ],
    out_specs=pl.BlockSpec((N // 5, H), lambda i: (i, 0)),
    out_shape=jax.ShapeDtypeStruct((N, H), jnp.float32))

_final_call = pl.pallas_call(
    _final_body, out_shape=jax.ShapeDtypeStruct((G, LATENT), jnp.float32))


_MESH = plsc.VectorSubcoreMesh(core_axis_name="c", subcore_axis_name="s")


@functools.partial(
    pl.kernel,
    out_type=jax.ShapeDtypeStruct((E, 16), jnp.float32),
    mesh=_MESH,
    scratch_types=[
        pltpu.VMEM((B,), jnp.int32),
        pltpu.VMEM((B,), jnp.int32),
        pltpu.VMEM((B, 128), jnp.float32),
        pltpu.VMEM((B, 128), jnp.float32),
        pltpu.VMEM((B, 16), jnp.float32),
        pltpu.SemaphoreType.DMA,
    ],
)
def _dist_kernel(pxyz_hbm, row_hbm, col_hbm, out_hbm,
                 ridx_v, cidx_v, pr_v, pc_v, sq_v, sem):
    c = lax.axis_index("c")
    s = lax.axis_index("s")
    tile_base = c * EPC + s * EPT

    def body(t, carry):
        base = tile_base + t * B
        pltpu.sync_copy(row_hbm.at[pl.ds(base, B)], ridx_v)
        pltpu.sync_copy(col_hbm.at[pl.ds(base, B)], cidx_v)
        pltpu.async_copy(pxyz_hbm.at[ridx_v], pr_v, sem).wait()
        pltpu.async_copy(pxyz_hbm.at[cidx_v], pc_v, sem).wait()

        def inner(e, c2):
            dx = pr_v[e, pl.ds(0, 16)] - pc_v[e, pl.ds(0, 16)]
            dy = pr_v[e, pl.ds(16, 16)] - pc_v[e, pl.ds(16, 16)]
            dz = pr_v[e, pl.ds(32, 16)] - pc_v[e, pl.ds(32, 16)]
            sq_v[e, :] = dx * dx + dy * dy + dz * dz + 1e-12
            return c2

        lax.fori_loop(0, B, inner, 0)
        pltpu.sync_copy(sq_v, out_hbm.at[pl.ds(base, B)])
        return carry

    lax.fori_loop(0, NB, body, 0)


@functools.partial(
    pl.kernel,
    out_type=jax.ShapeDtypeStruct((NC * N, 128), jnp.float32),
    mesh=_MESH,
    scratch_types=[
        pltpu.VMEM((1, B), jnp.int32),
        pltpu.VMEM((B,), jnp.int32),
        pltpu.VMEM((B, 16), jnp.float32),
        pltpu.VMEM((B, 2 * H), jnp.float32),
        pltpu.VMEM((B, 128), jnp.float32),
        pltpu.VMEM_SHARED((N, 128), jnp.float32),
        pltpu.SemaphoreType.DMA,
    ],
)
def _edge_kernel(r_hbm, row_hbm, col_hbm, dist_hbm, zero_hbm, out_hbm,
                 row_v, col_v, dist_v, rows_v, msg_v, agg_sh, sem):
    c = lax.axis_index("c")
    s = lax.axis_index("s")
    tile_base = c * EPC + s * EPT
    @pl.when(s < NS - 1)
    def _():
        pltpu.sync_copy(zero_hbm.at[pl.ds(s * 640, 640)],
                        agg_sh.at[pl.ds(s * 640, 640)])

    @pl.when(s == NS - 1)
    def _():
        pltpu.sync_copy(zero_hbm.at[pl.ds(9600, 400)],
                        agg_sh.at[pl.ds(9600, 400)])

    plsc.subcore_barrier()

    def body(t, carry):
        base = tile_base + t * B
        pltpu.sync_copy(row_hbm.at[pl.ds(base, B)], row_v.at[0])
        pltpu.sync_copy(col_hbm.at[pl.ds(base, B)], col_v)
        pltpu.sync_copy(dist_hbm.at[pl.ds(base, B)], dist_v)
        pltpu.async_copy(r_hbm.at[col_v], rows_v, sem).wait()

        def inner(e, c2):
            dv = dist_v[e, :]
            for j in range(H // 16):
                p = rows_v[e, pl.ds(j * 16, 16)]
                q = rows_v[e, pl.ds(H + j * 16, 16)]
                msg_v[e, pl.ds(j * 16, 16)] = dv * p + q
            return c2

        lax.fori_loop(0, B, inner, 0)
        pltpu.sync_copy(msg_v, agg_sh.at[row_v.at[0]], add=True)
        return carry

    lax.fori_loop(0, NB, body, 0)
    plsc.subcore_barrier()

    @pl.when(s < NS - 1)
    def _():
        pltpu.sync_copy(agg_sh.at[pl.ds(s * 640, 640)],
                        out_hbm.at[pl.ds(c * N + s * 640, 640)])

    @pl.when(s == NS - 1)
    def _():
        pltpu.sync_copy(agg_sh.at[pl.ds(9600, 400)],
                        out_hbm.at[pl.ds(c * N + 9600, 400)])


def kernel(x, pos, edge_index, batch, W_init, W_msg_tp_0, W_msg_lin_0,
           W_upd_tp_0, W_upd_lin_0, W_msg_tp_1, W_msg_lin_1, W_upd_tp_1,
           W_upd_lin_1, W_final):
    row = edge_index[0]
    col = edge_index[1]
    zero_nh = jnp.zeros((N, 128), jnp.float32)
    pxyz = jnp.concatenate(
        [jnp.broadcast_to(pos[:, i:i + 1], (N, 16)) for i in range(3)]
        + [jnp.zeros((N, 80), jnp.float32)], axis=1)

    sq = _dist_kernel(pxyz, row, col)
    distb = _sqrt_call(sq)

    h = _init_call(x, W_init)
    for (wtp, wlin, wupd, wulin) in (
            (W_msg_tp_0, W_msg_lin_0, W_upd_tp_0, W_upd_lin_0),
            (W_msg_tp_1, W_msg_lin_1, W_upd_tp_1, W_upd_lin_1)):
        r = _pq_call(h, wtp, wlin)
        agg2 = _edge_kernel(r, row, col, distb, zero_nh).reshape(NC, N, 128)
        wt = jnp.transpose(wupd, (1, 0, 2))
        h = _upd_call(h, agg2, wt, wulin)
    return _final_call(h, batch.reshape(1, N), W_final)


# speedup vs baseline: 2.3146x; 1.2440x over previous
import functools
import jax
import jax.numpy as jnp
from jax import lax
from jax.experimental import pallas as pl
from jax.experimental.pallas import tpu as pltpu
from jax.experimental.pallas import tpu_sc as plsc

N = 10000
E = 160000
DIN = 128
H = 64
G = 16
LATENT = 128

NC = 2
NS = 16
EPC = E // NC
EPT = EPC // NS
B = 40
NB = EPT // B


def _init_body(x_ref, w_ref, o_ref):
    o_ref[...] = jnp.dot(x_ref[...], w_ref[...],
                         preferred_element_type=jnp.float32) / jnp.sqrt(128.0)


def _pq_body(h_ref, wtp_ref, wlin_ref, o_ref):
    h = h_ref[...]
    p = jnp.dot(h, wtp_ref[...], preferred_element_type=jnp.float32) / 8.0
    q = jnp.dot(h, wlin_ref[...], preferred_element_type=jnp.float32) / 8.0
    o_ref[...] = jnp.concatenate([p, q], axis=1)


def _sqrt_body(s_ref, o_ref):
    o_ref[...] = jnp.sqrt(s_ref[...])


def _upd_body(h_ref, agg2_ref, wt_ref, wlin_ref, o_ref):
    h = h_ref[...]
    agg = agg2_ref[0][:, :H] + agg2_ref[1][:, :H]
    acc = h + jnp.dot(h, wlin_ref[...], preferred_element_type=jnp.float32) / 8.0
    aggs = agg / 64.0
    for j in range(H):
        hw = jnp.dot(h, wt_ref[j], preferred_element_type=jnp.float32)
        acc = acc + aggs[:, j:j + 1] * hw
    o_ref[...] = acc


def _final_body(h_ref, b_ref, wf_ref, o_ref):
    h = h_ref[...]
    b = b_ref[...]
    seg = jax.lax.broadcasted_iota(jnp.int32, (G, N), 0)
    mask = (b == seg).astype(jnp.float32)
    sums = jnp.dot(mask, h, preferred_element_type=jnp.float32)
    counts = jnp.sum(mask, axis=1, keepdims=True)
    mean = sums / jnp.maximum(counts, 1.0)
    o_ref[...] = jnp.dot(mean, wf_ref[...],
                         preferred_element_type=jnp.float32) / 8.0


_init_call = pl.pallas_call(
    _init_body, out_shape=jax.ShapeDtypeStruct((N, H), jnp.float32))

_pq_call = pl.pallas_call(
    _pq_body, out_shape=jax.ShapeDtypeStruct((N, 2 * H), jnp.float32))

_sqrt_call = pl.pallas_call(
    _sqrt_body,
    grid=(20,),
    in_specs=[pl.BlockSpec((E // 20, 16), lambda i: (i, 0))],
    out_specs=pl.BlockSpec((E // 20, 16), lambda i: (i, 0)),
    out_shape=jax.ShapeDtypeStruct((E, 16), jnp.float32))

_upd_call = pl.pallas_call(
    _upd_body,
    grid=(5,),
    in_specs=[
        pl.BlockSpec((N // 5, H), lambda i: (i, 0)),
        pl.BlockSpec((NC, N // 5, 128), lambda i: (0, i, 0)),
        pl.BlockSpec((H, H, H), lambda i: (0, 0, 0)),
        pl.BlockSpec((H, H), lambda i: (0, 0)),
    ],
    out_specs=pl.BlockSpec((N // 5, H), lambda i: (i, 0)),
    out_shape=jax.ShapeDtypeStruct((N, H), jnp.float32))

_final_call = pl.pallas_call(
    _final_body, out_shape=jax.ShapeDtypeStruct((G, LATENT), jnp.float32))


_MESH = plsc.VectorSubcoreMesh(core_axis_name="c", subcore_axis_name="s")


@functools.partial(
    pl.kernel,
    out_type=jax.ShapeDtypeStruct((E, 16), jnp.float32),
    mesh=_MESH,
    scratch_types=[
        pltpu.VMEM((B,), jnp.int32),
        pltpu.VMEM((B,), jnp.int32),
        pltpu.VMEM((B, 128), jnp.float32),
        pltpu.VMEM((B, 128), jnp.float32),
        pltpu.VMEM((B, 16), jnp.float32),
        pltpu.SemaphoreType.DMA,
    ],
)
def _dist_kernel(pxyz_hbm, row_hbm, col_hbm, out_hbm,
                 ridx_v, cidx_v, pr_v, pc_v, sq_v, sem):
    c = lax.axis_index("c")
    s = lax.axis_index("s")
    tile_base = c * EPC + s * EPT

    def body(t, carry):
        base = tile_base + t * B
        pltpu.sync_copy(row_hbm.at[pl.ds(base, B)], ridx_v)
        pltpu.sync_copy(col_hbm.at[pl.ds(base, B)], cidx_v)
        cp1 = pltpu.async_copy(pxyz_hbm.at[ridx_v], pr_v, sem)
        cp2 = pltpu.async_copy(pxyz_hbm.at[cidx_v], pc_v, sem)
        cp1.wait()
        cp2.wait()

        def inner(e, c2):
            dx = pr_v[e, pl.ds(0, 16)] - pc_v[e, pl.ds(0, 16)]
            dy = pr_v[e, pl.ds(16, 16)] - pc_v[e, pl.ds(16, 16)]
            dz = pr_v[e, pl.ds(32, 16)] - pc_v[e, pl.ds(32, 16)]
            sq_v[e, :] = dx * dx + dy * dy + dz * dz + 1e-12
            return c2

        lax.fori_loop(0, B, inner, 0)
        pltpu.sync_copy(sq_v, out_hbm.at[pl.ds(base, B)])
        return carry

    lax.fori_loop(0, NB, body, 0)


@functools.partial(
    pl.kernel,
    out_type=jax.ShapeDtypeStruct((NC * N, 128), jnp.float32),
    mesh=_MESH,
    scratch_types=[
        pltpu.VMEM((1, B), jnp.int32),
        pltpu.VMEM((B,), jnp.int32),
        pltpu.VMEM((B, 16), jnp.float32),
        pltpu.VMEM((B, 2 * H), jnp.float32),
        pltpu.VMEM((B, 128), jnp.float32),
        pltpu.VMEM_SHARED((N, 128), jnp.float32),
        pltpu.SemaphoreType.DMA,
    ],
)
def _edge_kernel(r_hbm, row_hbm, col_hbm, dist_hbm, zero_hbm, out_hbm,
                 row_v, col_v, dist_v, rows_v, msg_v, agg_sh, sem):
    c = lax.axis_index("c")
    s = lax.axis_index("s")
    tile_base = c * EPC + s * EPT
    @pl.when(s < NS - 1)
    def _():
        pltpu.sync_copy(zero_hbm.at[pl.ds(s * 640, 640)],
                        agg_sh.at[pl.ds(s * 640, 640)])

    @pl.when(s == NS - 1)
    def _():
        pltpu.sync_copy(zero_hbm.at[pl.ds(9600, 400)],
                        agg_sh.at[pl.ds(9600, 400)])

    plsc.subcore_barrier()

    def body(t, carry):
        base = tile_base + t * B
        pltpu.sync_copy(col_hbm.at[pl.ds(base, B)], col_v)
        cp = pltpu.async_copy(r_hbm.at[col_v], rows_v, sem)
        pltpu.sync_copy(row_hbm.at[pl.ds(base, B)], row_v.at[0])
        pltpu.sync_copy(dist_hbm.at[pl.ds(base, B)], dist_v)
        cp.wait()

        def inner(e, c2):
            dv = dist_v[e, :]
            for j in range(H // 16):
                p = rows_v[e, pl.ds(j * 16, 16)]
                q = rows_v[e, pl.ds(H + j * 16, 16)]
                msg_v[e, pl.ds(j * 16, 16)] = dv * p + q
            return c2

        lax.fori_loop(0, B, inner, 0)
        pltpu.sync_copy(msg_v, agg_sh.at[row_v.at[0]], add=True)
        return carry

    lax.fori_loop(0, NB, body, 0)
    plsc.subcore_barrier()

    @pl.when(s < NS - 1)
    def _():
        pltpu.sync_copy(agg_sh.at[pl.ds(s * 640, 640)],
                        out_hbm.at[pl.ds(c * N + s * 640, 640)])

    @pl.when(s == NS - 1)
    def _():
        pltpu.sync_copy(agg_sh.at[pl.ds(9600, 400)],
                        out_hbm.at[pl.ds(c * N + 9600, 400)])


def kernel(x, pos, edge_index, batch, W_init, W_msg_tp_0, W_msg_lin_0,
           W_upd_tp_0, W_upd_lin_0, W_msg_tp_1, W_msg_lin_1, W_upd_tp_1,
           W_upd_lin_1, W_final):
    row = edge_index[0]
    col = edge_index[1]
    zero_nh = jnp.zeros((N, 128), jnp.float32)
    pxyz = jnp.concatenate(
        [jnp.broadcast_to(pos[:, i:i + 1], (N, 16)) for i in range(3)]
        + [jnp.zeros((N, 80), jnp.float32)], axis=1)

    sq = _dist_kernel(pxyz, row, col)
    distb = _sqrt_call(sq)

    h = _init_call(x, W_init)
    for (wtp, wlin, wupd, wulin) in (
            (W_msg_tp_0, W_msg_lin_0, W_upd_tp_0, W_upd_lin_0),
            (W_msg_tp_1, W_msg_lin_1, W_upd_tp_1, W_upd_lin_1)):
        r = _pq_call(h, wtp, wlin)
        agg2 = _edge_kernel(r, row, col, distb, zero_nh).reshape(NC, N, 128)
        wt = jnp.transpose(wupd, (1, 0, 2))
        h = _upd_call(h, agg2, wt, wulin)
    return _final_call(h, batch.reshape(1, N), W_final)
